# Initial kernel scaffold; baseline (speedup 1.0000x reference)
import jax
import jax.numpy as jnp
from jax.experimental import pallas as pl


def kernel(x, edge_index, W1, b1, W2, b2, W3, b3):
    raise NotImplementedError("write your pallas kernel here")


# speedup vs baseline: 4.1056x; 4.1056x over previous
import functools

import jax
import jax.numpy as jnp
from jax import lax
from jax.experimental import pallas as pl
from jax.experimental.pallas import tpu as pltpu
from jax.experimental.pallas import tpu_sc as plsc

N = 10000
E = 320000
F_IN = 128
H = 128
C = 47

NC = 2
NS = 16
K = 128

W_E = ((E // (NC * NS) + K - 1) // K) * K
E_PAD = W_E * NC * NS
N_ACC = 10112
TRASH = N


def _make_segsum(Hf):
    mesh = plsc.VectorSubcoreMesh(core_axis_name="c", subcore_axis_name="s")

    @functools.partial(
        pl.kernel,
        out_type=jax.ShapeDtypeStruct((NC * N_ACC, Hf), jnp.float32),
        mesh=mesh,
        scratch_types=[
            pltpu.VMEM((K,), jnp.int32),
            pltpu.VMEM((K,), jnp.int32),
            pltpu.VMEM((K, Hf), jnp.float32),
            pltpu.VMEM_SHARED((N_ACC, Hf), jnp.float32),
            pltpu.SemaphoreType.DMA,
        ],
        compiler_params=pltpu.CompilerParams(use_tc_tiling_on_sc=False),
    )
    def segsum(y_hbm, src_hbm, dst_hbm, zeros_hbm, out_hbm,
               src_v, dst_v, rows_v, acc_sh, sem):
        c = lax.axis_index("c")
        s = lax.axis_index("s")
        wid = c * NS + s
        zr = N_ACC // NS
        pltpu.sync_copy(zeros_hbm.at[pl.ds(s * zr, zr)],
                        acc_sh.at[pl.ds(s * zr, zr)])
        plsc.subcore_barrier()
        base = wid * W_E

        def body(j, carry):
            off = base + j * K
            pltpu.sync_copy(src_hbm.at[pl.ds(off, K)], src_v)
            pltpu.sync_copy(dst_hbm.at[pl.ds(off, K)], dst_v)
            pltpu.async_copy(y_hbm.at[src_v], rows_v, sem).wait()
            pltpu.sync_copy(rows_v, acc_sh.at[dst_v], add=True)
            return carry

        lax.fori_loop(0, W_E // K, body, 0)
        plsc.subcore_barrier()
        nr = N_ACC // NS
        pltpu.sync_copy(acc_sh.at[pl.ds(s * nr, nr)],
                        out_hbm.at[pl.ds(c * N_ACC + s * nr, nr)])

    return segsum


_segsum_128 = _make_segsum(H)
_segsum_48 = _make_segsum(48)


def _dense_pq(h, wt, wb, b2d, ho, relu_in=None):
    n = h.shape[0]
    bn = 1000
    fin = h.shape[1]

    if relu_in is None:
        def body(h_ref, wt_ref, wb_ref, b_ref, p_ref, q_ref):
            hb = h_ref[...]
            p_ref[...] = jnp.dot(hb, wt_ref[...],
                                 preferred_element_type=jnp.float32) + b_ref[...]
            q_ref[...] = jnp.dot(hb, wb_ref[...],
                                 preferred_element_type=jnp.float32)

        in_specs = [
            pl.BlockSpec((bn, fin), lambda i: (i, 0)),
            pl.BlockSpec((fin, ho), lambda i: (0, 0)),
            pl.BlockSpec((fin, ho), lambda i: (0, 0)),
            pl.BlockSpec((1, ho), lambda i: (0, 0)),
        ]
        args = (h, wt, wb, b2d)
    else:
        p_prev, a0, a1 = relu_in

        def body(pp_ref, a0_ref, a1_ref, wt_ref, wb_ref, b_ref, p_ref, q_ref):
            hb = jnp.maximum(pp_ref[...] + a0_ref[...] + a1_ref[...], 0.0)
            p_ref[...] = jnp.dot(hb, wt_ref[...],
                                 preferred_element_type=jnp.float32) + b_ref[...]
            q_ref[...] = jnp.dot(hb, wb_ref[...],
                                 preferred_element_type=jnp.float32)

        in_specs = [
            pl.BlockSpec((bn, fin), lambda i: (i, 0)),
            pl.BlockSpec((bn, fin), lambda i: (i, 0)),
            pl.BlockSpec((bn, fin), lambda i: (i, 0)),
            pl.BlockSpec((fin, ho), lambda i: (0, 0)),
            pl.BlockSpec((fin, ho), lambda i: (0, 0)),
            pl.BlockSpec((1, ho), lambda i: (0, 0)),
        ]
        args = (p_prev, a0, a1, wt, wb, b2d)

    return pl.pallas_call(
        body,
        grid=(n // bn,),
        in_specs=in_specs,
        out_specs=[
            pl.BlockSpec((bn, ho), lambda i: (i, 0)),
            pl.BlockSpec((bn, ho), lambda i: (i, 0)),
        ],
        out_shape=[
            jax.ShapeDtypeStruct((n, ho), jnp.float32),
            jax.ShapeDtypeStruct((n, ho), jnp.float32),
        ],
    )(*args)


def _final_logsoftmax(p3, a0, a1):
    n = p3.shape[0]
    bn = 1000
    w = p3.shape[1]

    def body(p_ref, a0_ref, a1_ref, o_ref):
        z = p_ref[...] + a0_ref[...] + a1_ref[...]
        col = lax.broadcasted_iota(jnp.int32, (bn, w), 1)
        valid = col < C
        zm = jnp.where(valid, z, -jnp.inf)
        m = jnp.max(zm, axis=-1, keepdims=True)
        e = jnp.where(valid, jnp.exp(z - m), 0.0)
        lse = jnp.log(jnp.sum(e, axis=-1, keepdims=True)) + m
        o_ref[...] = (z - lse)[:, :C]

    return pl.pallas_call(
        body,
        grid=(n // bn,),
        in_specs=[
            pl.BlockSpec((bn, w), lambda i: (i, 0)),
            pl.BlockSpec((bn, w), lambda i: (i, 0)),
            pl.BlockSpec((bn, w), lambda i: (i, 0)),
        ],
        out_specs=pl.BlockSpec((bn, C), lambda i: (i, 0)),
        out_shape=jax.ShapeDtypeStruct((n, C), jnp.float32),
    )(p3, a0, a1)


def kernel(x, edge_index, W1, b1, W2, b2, W3, b3):
    src = edge_index[0]
    dst = edge_index[1]
    pad = E_PAD - E
    src_p = jnp.concatenate([src, jnp.zeros((pad,), jnp.int32)])
    dst_p = jnp.concatenate([dst, jnp.full((pad,), TRASH, jnp.int32)])
    zeros128 = jnp.zeros((N_ACC, H), jnp.float32)
    zeros48 = jnp.zeros((N_ACC, 48), jnp.float32)

    p1, q1 = _dense_pq(x, W1[:F_IN], W1[F_IN:], b1.reshape(1, -1), H)
    a1 = _segsum_128(q1, src_p, dst_p, zeros128)
    a1_0, a1_1 = a1[:N], a1[N_ACC:N_ACC + N]

    p2, q2 = _dense_pq(p1, W2[:H], W2[H:], b2.reshape(1, -1), H,
                       relu_in=(p1, a1_0, a1_1))
    a2 = _segsum_128(q2, src_p, dst_p, zeros128)
    a2_0, a2_1 = a2[:N], a2[N_ACC:N_ACC + N]

    w3t = jnp.pad(W3[:H], ((0, 0), (0, 1)))
    w3b = jnp.pad(W3[H:], ((0, 0), (0, 1)))
    b3p = jnp.pad(b3, (0, 1)).reshape(1, -1)
    p3, q3 = _dense_pq(p2, w3t, w3b, b3p, 48,
                       relu_in=(p2, a2_0, a2_1))
    a3 = _segsum_48(q3, src_p, dst_p, zeros48)

    return _final_logsoftmax(p3, a3[:N], a3[N_ACC:N_ACC + N])
